# baseline (speedup 1.0000x reference)
import jax
import jax.numpy as jnp
from jax import lax
from jax.experimental import pallas as pl
from jax.experimental.pallas import tpu as pltpu
from jax.experimental.pallas import tpu_sc as plsc

N = 10000
E = 320000
D = 128
R = 8

NC, NS, LANES = 2, 16, 16
NW = NC * NS
K = 32
K2 = 2 * K
EPT = 10240
CH = EPT // K
CH0 = 224
CH1 = 2 * CH - CH0
EP = EPT * NW
G = EP // K
NPAD = 10240
DR = 256
NT = NPAD + DR
ST = NT // NS
BN = 1000
NB = N // BN


def _mm_body(x_ref, w_ref, o_ref):
    for r in range(R + 1):
        o_ref[r] = jnp.dot(x_ref[...], w_ref[r],
                           preferred_element_type=jnp.float32)


def _idx_body(src_ref, et_ref, dst_ref, w_ref, meta_ref, wrr_ref):
    dst = dst_ref[...]
    meta_ref[:, 0, 0:K] = (et_ref[...] + 1) * N + src_ref[...]
    meta_ref[:, 0, K:K2] = jnp.zeros(dst.shape, jnp.int32)
    meta_ref[:, 1, 0:K] = dst
    meta_ref[:, 1, K:K2] = NPAD + (dst >> 6)
    wrr_ref[:, :, 0:LANES] = jnp.broadcast_to(
        w_ref[...][:, :, None], (dst.shape[0], K, LANES))
    wrr_ref[:, :, LANES:] = jnp.broadcast_to(
        (dst & 63).astype(jnp.float32)[:, :, None], (dst.shape[0], K, LANES))


def _finish_body(root_ref, a0_ref, a1_ref, d0_ref, d1_ref, o_ref):
    deg = jnp.maximum(d0_ref[...] + d1_ref[...], 1.0)
    o_ref[...] = (root_ref[...] + a0_ref[0] + a1_ref[0]) / deg


def _hi16():
    return lax.shift_right_logical(
        lax.iota(jnp.int32, 16), jnp.full((16,), 1, jnp.int32)
    ).astype(jnp.float32)


def _sc_body(table, meta3, wrr, accp,
             acc_sh, meta_v, wrr_v, rows_v, sem_g, sem_a, sem_m):
    c = lax.axis_index("c")
    s = lax.axis_index("s")
    nch = jnp.where(c == 0, CH0, CH1)
    base = jnp.where(c == 0, s * CH0, NS * CH0 + s * CH1)
    hi = _hi16()

    def fill_body(i, carry):
        for f in range(D // LANES):
            rows_v[0][i, pl.ds(f * LANES, LANES)] = jnp.zeros((LANES,), jnp.float32)
        return carry
    lax.fori_loop(0, K2, fill_body, 0)

    for q in range(ST // K2):
        pltpu.async_copy(rows_v[0], acc_sh.at[pl.ds(s * ST + q * K2, K2)], sem_a[0])
    pltpu.async_copy(rows_v[0].at[pl.ds(0, ST % K2)],
                     acc_sh.at[pl.ds(s * ST + (ST // K2) * K2, ST % K2)], sem_a[0])
    for q in range(ST // K2):
        pltpu.make_async_copy(table.at[pl.ds(0, K2)], rows_v[0], sem_a[0]).wait()
    pltpu.make_async_copy(table.at[pl.ds(0, ST % K2)],
                          rows_v[0].at[pl.ds(0, ST % K2)], sem_a[0]).wait()

    plsc.subcore_barrier()

    def stage_issue(j, m):
        pltpu.async_copy(meta3.at[base + j], meta_v[m], sem_m[m])
        pltpu.async_copy(wrr.at[pl.ds((base + j) * K, K)], wrr_v[m], sem_m[m])

    def stage_wait(m):
        pltpu.make_async_copy(meta3.at[0], meta_v[m], sem_m[m]).wait()
        pltpu.make_async_copy(wrr.at[pl.ds(0, K)], wrr_v[m], sem_m[m]).wait()

    def deg_build(m, b):
        def deg_body(e, inner):
            gsp = wrr_v[m][e, pl.ds(LANES, LANES)]
            for f in range(D // LANES):
                cf = hi + float(8 * f)
                rows_v[b][K + e, pl.ds(f * LANES, LANES)] = jnp.where(
                    gsp == cf, 1.0, 0.0)
            return inner
        lax.fori_loop(0, K, deg_body, 0)

    def gather(m, b):
        pltpu.async_copy(table.at[meta_v[m].at[0, pl.ds(0, K)]],
                         rows_v[b].at[pl.ds(0, K)], sem_g[b])

    stage_issue(0, 0)
    stage_wait(0)
    deg_build(0, 0)
    gather(0, 0)
    stage_issue(1, 1)

    def quad_body(jj, carry):
        for u in range(4):
            j = jj * 4 + u
            b = u % 2
            nb = 1 - b
            m1 = (u + 1) % 4
            m2 = (u + 2) % 4

            @pl.when(j + 2 < nch)
            def _():
                stage_issue(j + 2, m2)

            @pl.when(j >= 1)
            def _():
                pltpu.make_async_copy(
                    table.at[pl.ds(0, K2)], rows_v[nb], sem_a[nb]).wait()

            @pl.when(j + 1 < nch)
            def _():
                stage_wait(m1)
                gather(m1, nb)

            pltpu.make_async_copy(
                table.at[pl.ds(0, K)], rows_v[b].at[pl.ds(0, K)], sem_g[b]).wait()

            def fused_body(e, inner):
                wsp = wrr_v[u][e, pl.ds(0, LANES)]
                for f in range(D // LANES):
                    sl = pl.ds(f * LANES, LANES)
                    rows_v[b][e, sl] = rows_v[b][e, sl] * wsp
                gsp = wrr_v[m1][e, pl.ds(LANES, LANES)]
                for f in range(D // LANES):
                    cf = hi + float(8 * f)
                    rows_v[nb][K + e, pl.ds(f * LANES, LANES)] = jnp.where(
                        gsp == cf, 1.0, 0.0)
                return inner
            lax.fori_loop(0, K, fused_body, 0)

            pltpu.async_copy(rows_v[b], acc_sh.at[meta_v[u].at[1]], sem_a[b],
                             add=True)
        return carry
    lax.fori_loop(0, nch // 4, quad_body, 0)

    pltpu.make_async_copy(table.at[pl.ds(0, K2)], rows_v[1], sem_a[1]).wait()

    plsc.subcore_barrier()

    NQ = ST // K2
    for q in range(NQ + 1):
        b = q % 2
        rows = K2 if q < NQ else ST % K2
        sl = pl.ds(s * ST + q * K2, rows)
        buf = rows_v[b] if rows == K2 else rows_v[b].at[pl.ds(0, rows)]
        if q >= 2:
            prev = K2 if q - 2 < NQ else ST % K2
            pbuf = rows_v[b] if prev == K2 else rows_v[b].at[pl.ds(0, prev)]
            pltpu.make_async_copy(table.at[pl.ds(0, prev)], pbuf, sem_a[b]).wait()
        pltpu.sync_copy(acc_sh.at[sl], buf)
        pltpu.async_copy(buf, accp.at[c, sl], sem_a[b])
    for q in [NQ - 1, NQ]:
        b = q % 2
        rows = K2 if q < NQ else ST % K2
        fbuf = rows_v[b] if rows == K2 else rows_v[b].at[pl.ds(0, rows)]
        pltpu.make_async_copy(table.at[pl.ds(0, rows)], fbuf, sem_a[b]).wait()


def kernel(x, edge_index, edge_type, num_nodes, edge_weight, W, W0):
    x = x.astype(jnp.float32)

    Wall = jnp.concatenate([W0[None], W], axis=0)
    BM = 1000
    table = pl.pallas_call(
        _mm_body,
        grid=(N // BM,),
        in_specs=[
            pl.BlockSpec((BM, D), lambda b: (b, 0)),
            pl.BlockSpec((R + 1, D, D), lambda b: (0, 0, 0)),
        ],
        out_specs=pl.BlockSpec((R + 1, BM, D), lambda b: (0, b, 0)),
        out_shape=jax.ShapeDtypeStruct((R + 1, N, D), jnp.float32),
    )(x, Wall).reshape((R + 1) * N, D)

    pad = EP - E
    src_p = jnp.pad(edge_index[0].astype(jnp.int32), (0, pad)).reshape(G, K)
    et_p = jnp.pad(edge_type.astype(jnp.int32), (0, pad)).reshape(G, K)
    dst2 = jnp.pad(edge_index[1].astype(jnp.int32), (0, pad),
                   constant_values=N).reshape(G, K)
    w2 = jnp.pad(edge_weight.astype(jnp.float32), (0, pad)).reshape(G, K)

    EB = 1024
    meta3, wrr3 = pl.pallas_call(
        _idx_body,
        grid=(G // EB,),
        in_specs=[
            pl.BlockSpec((EB, K), lambda i: (i, 0)),
            pl.BlockSpec((EB, K), lambda i: (i, 0)),
            pl.BlockSpec((EB, K), lambda i: (i, 0)),
            pl.BlockSpec((EB, K), lambda i: (i, 0)),
        ],
        out_specs=[
            pl.BlockSpec((EB, 2, K2), lambda i: (i, 0, 0)),
            pl.BlockSpec((EB, K, 2 * LANES), lambda i: (i, 0, 0)),
        ],
        out_shape=[
            jax.ShapeDtypeStruct((G, 2, K2), jnp.int32),
            jax.ShapeDtypeStruct((G, K, 2 * LANES), jnp.float32),
        ],
    )(src_p, et_p, dst2, w2)
    wrr = wrr3.reshape(EP, 2 * LANES)

    mesh = plsc.VectorSubcoreMesh(core_axis_name="c", subcore_axis_name="s",
                                  num_cores=NC, num_subcores=NS)
    accp = pl.kernel(
        _sc_body,
        out_type=jax.ShapeDtypeStruct((NC, NT, D), jnp.float32),
        mesh=mesh,
        scratch_types=[
            pltpu.VMEM_SHARED((NT, D), jnp.float32),
            [pltpu.VMEM((2, K2), jnp.int32)] * 4,
            [pltpu.VMEM((K, 2 * LANES), jnp.float32)] * 4,
            [pltpu.VMEM((K2, D), jnp.float32)] * 2,
            [pltpu.SemaphoreType.DMA] * 2,
            [pltpu.SemaphoreType.DMA] * 2,
            [pltpu.SemaphoreType.DMA] * 4,
        ],
    )(table, meta3, wrr)

    degx0 = accp[0, NPAD:NPAD + NPAD // 64].reshape(NPAD // 64, 64, 2)[
        :, :, 0:1].reshape(NPAD, 1)
    degx1 = accp[1, NPAD:NPAD + NPAD // 64].reshape(NPAD // 64, 64, 2)[
        :, :, 0:1].reshape(NPAD, 1)

    BF = 2000
    out = pl.pallas_call(
        _finish_body,
        grid=(N // BF,),
        in_specs=[
            pl.BlockSpec((BF, D), lambda b: (b, 0)),
            pl.BlockSpec((1, BF, D), lambda b: (0, b, 0)),
            pl.BlockSpec((1, BF, D), lambda b: (1, b, 0)),
            pl.BlockSpec((BF, 1), lambda b: (b, 0)),
            pl.BlockSpec((BF, 1), lambda b: (b, 0)),
        ],
        out_specs=pl.BlockSpec((BF, D), lambda b: (b, 0)),
        out_shape=jax.ShapeDtypeStruct((N, D), jnp.float32),
    )(table, accp, accp, degx0, degx1)
    return out


# speedup vs baseline: 1.1711x; 1.1711x over previous
import jax
import jax.numpy as jnp
from jax import lax
from jax.experimental import pallas as pl
from jax.experimental.pallas import tpu as pltpu
from jax.experimental.pallas import tpu_sc as plsc

N = 10000
E = 320000
D = 128
R = 8

NC, NS, LANES = 2, 16, 16
NW = NC * NS
K = 32
K2 = 2 * K
EPT = 10240
CH = EPT // K
CH0 = 416
CH1 = 2 * CH - CH0
EP = EPT * NW
G = EP // K
NPAD = 10240
DR = 256
NT = NPAD + DR
ST = NT // NS
BN = 1000
NB = N // BN


def _mm_body(x_ref, w_ref, o_ref):
    for r in range(R + 1):
        o_ref[r] = jnp.dot(x_ref[...], w_ref[r],
                           preferred_element_type=jnp.float32)


def _idx_body(src_ref, et_ref, dst_ref, w_ref, meta_ref, wrr_ref):
    dst = dst_ref[...]
    meta_ref[:, 0, 0:K] = (et_ref[...] + 1) * N + src_ref[...]
    meta_ref[:, 0, K:K2] = jnp.zeros(dst.shape, jnp.int32)
    meta_ref[:, 1, 0:K] = dst
    meta_ref[:, 1, K:K2] = NPAD + (dst >> 6)
    wrr_ref[:, :, 0:LANES] = jnp.broadcast_to(
        w_ref[...][:, :, None], (dst.shape[0], K, LANES))
    wrr_ref[:, :, LANES:] = jnp.broadcast_to(
        (dst & 63).astype(jnp.float32)[:, :, None], (dst.shape[0], K, LANES))


def _finish_body(root_ref, a0_ref, a1_ref, d0_ref, d1_ref, o_ref):
    deg = jnp.maximum(d0_ref[...] + d1_ref[...], 1.0)
    o_ref[...] = (root_ref[...] + a0_ref[0] + a1_ref[0]) / deg


def _hi16():
    return lax.shift_right_logical(
        lax.iota(jnp.int32, 16), jnp.full((16,), 1, jnp.int32)
    ).astype(jnp.float32)


def _sc_body(table, meta3, wrr, accp,
             acc_sh, meta_v, wrr_v, rows_v, sem_g, sem_a, sem_m):
    c = lax.axis_index("c")
    s = lax.axis_index("s")
    nch = jnp.where(c == 0, CH0, CH1)
    base = jnp.where(c == 0, s * CH0, NS * CH0 + s * CH1)
    hi = _hi16()

    def fill_body(i, carry):
        for f in range(D // LANES):
            rows_v[0][i, pl.ds(f * LANES, LANES)] = jnp.zeros((LANES,), jnp.float32)
        return carry
    lax.fori_loop(0, K2, fill_body, 0)

    for q in range(ST // K2):
        pltpu.async_copy(rows_v[0], acc_sh.at[pl.ds(s * ST + q * K2, K2)], sem_a[0])
    pltpu.async_copy(rows_v[0].at[pl.ds(0, ST % K2)],
                     acc_sh.at[pl.ds(s * ST + (ST // K2) * K2, ST % K2)], sem_a[0])
    for q in range(ST // K2):
        pltpu.make_async_copy(table.at[pl.ds(0, K2)], rows_v[0], sem_a[0]).wait()
    pltpu.make_async_copy(table.at[pl.ds(0, ST % K2)],
                          rows_v[0].at[pl.ds(0, ST % K2)], sem_a[0]).wait()

    plsc.subcore_barrier()

    def stage_issue(j, m):
        pltpu.async_copy(meta3.at[base + j], meta_v[m], sem_m[m])
        pltpu.async_copy(wrr.at[pl.ds((base + j) * K, K)], wrr_v[m], sem_m[m])

    def stage_wait(m):
        pltpu.make_async_copy(meta3.at[0], meta_v[m], sem_m[m]).wait()
        pltpu.make_async_copy(wrr.at[pl.ds(0, K)], wrr_v[m], sem_m[m]).wait()

    def deg_build(m, b):
        def deg_body(e, inner):
            gsp = wrr_v[m][e, pl.ds(LANES, LANES)]
            for f in range(D // LANES):
                cf = hi + float(8 * f)
                rows_v[b][K + e, pl.ds(f * LANES, LANES)] = jnp.where(
                    gsp == cf, 1.0, 0.0)
            return inner
        lax.fori_loop(0, K, deg_body, 0)

    def gather(m, b):
        pltpu.async_copy(table.at[meta_v[m].at[0, pl.ds(0, K)]],
                         rows_v[b].at[pl.ds(0, K)], sem_g[b])

    stage_issue(0, 0)
    stage_wait(0)
    deg_build(0, 0)
    gather(0, 0)
    stage_issue(1, 1)

    def quad_body(jj, carry):
        for u in range(4):
            j = jj * 4 + u
            b = u % 2
            nb = 1 - b
            m1 = (u + 1) % 4
            m2 = (u + 2) % 4

            @pl.when(j + 2 < nch)
            def _():
                stage_issue(j + 2, m2)

            @pl.when(j >= 1)
            def _():
                pltpu.make_async_copy(
                    table.at[pl.ds(0, K2)], rows_v[nb], sem_a[nb]).wait()

            @pl.when(j + 1 < nch)
            def _():
                stage_wait(m1)
                gather(m1, nb)

            pltpu.make_async_copy(
                table.at[pl.ds(0, K)], rows_v[b].at[pl.ds(0, K)], sem_g[b]).wait()

            def fused_body(e, inner):
                wsp = wrr_v[u][e, pl.ds(0, LANES)]
                for f in range(D // LANES):
                    sl = pl.ds(f * LANES, LANES)
                    rows_v[b][e, sl] = rows_v[b][e, sl] * wsp
                gsp = wrr_v[m1][e, pl.ds(LANES, LANES)]
                for f in range(D // LANES):
                    cf = hi + float(8 * f)
                    rows_v[nb][K + e, pl.ds(f * LANES, LANES)] = jnp.where(
                        gsp == cf, 1.0, 0.0)
                return inner
            lax.fori_loop(0, K, fused_body, 0)

            pltpu.async_copy(rows_v[b], acc_sh.at[meta_v[u].at[1]], sem_a[b],
                             add=True)
        return carry
    lax.fori_loop(0, nch // 4, quad_body, 0)

    pltpu.make_async_copy(table.at[pl.ds(0, K2)], rows_v[1], sem_a[1]).wait()

    plsc.subcore_barrier()

    NQ = ST // K2
    for q in range(NQ + 1):
        b = q % 2
        rows = K2 if q < NQ else ST % K2
        sl = pl.ds(s * ST + q * K2, rows)
        buf = rows_v[b] if rows == K2 else rows_v[b].at[pl.ds(0, rows)]
        if q >= 2:
            prev = K2 if q - 2 < NQ else ST % K2
            pbuf = rows_v[b] if prev == K2 else rows_v[b].at[pl.ds(0, prev)]
            pltpu.make_async_copy(table.at[pl.ds(0, prev)], pbuf, sem_a[b]).wait()
        pltpu.sync_copy(acc_sh.at[sl], buf)
        pltpu.async_copy(buf, accp.at[c, sl], sem_a[b])
    for q in [NQ - 1, NQ]:
        b = q % 2
        rows = K2 if q < NQ else ST % K2
        fbuf = rows_v[b] if rows == K2 else rows_v[b].at[pl.ds(0, rows)]
        pltpu.make_async_copy(table.at[pl.ds(0, rows)], fbuf, sem_a[b]).wait()


def kernel(x, edge_index, edge_type, num_nodes, edge_weight, W, W0):
    x = x.astype(jnp.float32)

    Wall = jnp.concatenate([W0[None], W], axis=0)
    BM = 1000
    table = pl.pallas_call(
        _mm_body,
        grid=(N // BM,),
        in_specs=[
            pl.BlockSpec((BM, D), lambda b: (b, 0)),
            pl.BlockSpec((R + 1, D, D), lambda b: (0, 0, 0)),
        ],
        out_specs=pl.BlockSpec((R + 1, BM, D), lambda b: (0, b, 0)),
        out_shape=jax.ShapeDtypeStruct((R + 1, N, D), jnp.float32),
    )(x, Wall).reshape((R + 1) * N, D)

    pad = EP - E
    src_p = jnp.pad(edge_index[0].astype(jnp.int32), (0, pad)).reshape(G, K)
    et_p = jnp.pad(edge_type.astype(jnp.int32), (0, pad)).reshape(G, K)
    dst2 = jnp.pad(edge_index[1].astype(jnp.int32), (0, pad),
                   constant_values=N).reshape(G, K)
    w2 = jnp.pad(edge_weight.astype(jnp.float32), (0, pad)).reshape(G, K)

    EB = 1024
    meta3, wrr3 = pl.pallas_call(
        _idx_body,
        grid=(G // EB,),
        in_specs=[
            pl.BlockSpec((EB, K), lambda i: (i, 0)),
            pl.BlockSpec((EB, K), lambda i: (i, 0)),
            pl.BlockSpec((EB, K), lambda i: (i, 0)),
            pl.BlockSpec((EB, K), lambda i: (i, 0)),
        ],
        out_specs=[
            pl.BlockSpec((EB, 2, K2), lambda i: (i, 0, 0)),
            pl.BlockSpec((EB, K, 2 * LANES), lambda i: (i, 0, 0)),
        ],
        out_shape=[
            jax.ShapeDtypeStruct((G, 2, K2), jnp.int32),
            jax.ShapeDtypeStruct((G, K, 2 * LANES), jnp.float32),
        ],
    )(src_p, et_p, dst2, w2)
    wrr = wrr3.reshape(EP, 2 * LANES)

    mesh = plsc.VectorSubcoreMesh(core_axis_name="c", subcore_axis_name="s",
                                  num_cores=NC, num_subcores=NS)
    accp = pl.kernel(
        _sc_body,
        out_type=jax.ShapeDtypeStruct((NC, NT, D), jnp.float32),
        mesh=mesh,
        scratch_types=[
            pltpu.VMEM_SHARED((NT, D), jnp.float32),
            [pltpu.VMEM((2, K2), jnp.int32)] * 4,
            [pltpu.VMEM((K, 2 * LANES), jnp.float32)] * 4,
            [pltpu.VMEM((K2, D), jnp.float32)] * 2,
            [pltpu.SemaphoreType.DMA] * 2,
            [pltpu.SemaphoreType.DMA] * 2,
            [pltpu.SemaphoreType.DMA] * 4,
        ],
    )(table, meta3, wrr)

    degx0 = accp[0, NPAD:NPAD + NPAD // 64].reshape(NPAD // 64, 64, 2)[
        :, :, 0:1].reshape(NPAD, 1)
    degx1 = accp[1, NPAD:NPAD + NPAD // 64].reshape(NPAD // 64, 64, 2)[
        :, :, 0:1].reshape(NPAD, 1)

    BF = 2000
    out = pl.pallas_call(
        _finish_body,
        grid=(N // BF,),
        in_specs=[
            pl.BlockSpec((BF, D), lambda b: (b, 0)),
            pl.BlockSpec((1, BF, D), lambda b: (0, b, 0)),
            pl.BlockSpec((1, BF, D), lambda b: (1, b, 0)),
            pl.BlockSpec((BF, 1), lambda b: (b, 0)),
            pl.BlockSpec((BF, 1), lambda b: (b, 0)),
        ],
        out_specs=pl.BlockSpec((BF, D), lambda b: (b, 0)),
        out_shape=jax.ShapeDtypeStruct((N, D), jnp.float32),
    )(table, accp, accp, degx0, degx1)
    return out


# speedup vs baseline: 1.1894x; 1.0157x over previous
import jax
import jax.numpy as jnp
from jax import lax
from jax.experimental import pallas as pl
from jax.experimental.pallas import tpu as pltpu
from jax.experimental.pallas import tpu_sc as plsc

N = 10000
E = 320000
D = 128
R = 8

NC, NS, LANES = 2, 16, 16
NW = NC * NS
K = 32
K2 = 2 * K
EPT = 10240
CH = EPT // K
CH0 = 480
CH1 = 2 * CH - CH0
EP = EPT * NW
G = EP // K
NPAD = 10240
DR = 256
NT = NPAD + DR
ST = NT // NS
BN = 1000
NB = N // BN


def _mm_body(x_ref, w_ref, o_ref):
    for r in range(R + 1):
        o_ref[r] = jnp.dot(x_ref[...], w_ref[r],
                           preferred_element_type=jnp.float32)


def _idx_body(src_ref, et_ref, dst_ref, w_ref, meta_ref, wrr_ref):
    dst = dst_ref[...]
    meta_ref[:, 0, 0:K] = (et_ref[...] + 1) * N + src_ref[...]
    meta_ref[:, 0, K:K2] = jnp.zeros(dst.shape, jnp.int32)
    meta_ref[:, 1, 0:K] = dst
    meta_ref[:, 1, K:K2] = NPAD + (dst >> 6)
    wrr_ref[:, :, 0:LANES] = jnp.broadcast_to(
        w_ref[...][:, :, None], (dst.shape[0], K, LANES))
    wrr_ref[:, :, LANES:] = jnp.broadcast_to(
        (dst & 63).astype(jnp.float32)[:, :, None], (dst.shape[0], K, LANES))


def _finish_body(root_ref, a0_ref, a1_ref, d0_ref, d1_ref, o_ref):
    deg = jnp.maximum(d0_ref[...] + d1_ref[...], 1.0)
    o_ref[...] = (root_ref[...] + a0_ref[0] + a1_ref[0]) / deg


def _hi16():
    return lax.shift_right_logical(
        lax.iota(jnp.int32, 16), jnp.full((16,), 1, jnp.int32)
    ).astype(jnp.float32)


def _sc_body(table, meta3, wrr, accp,
             acc_sh, meta_v, wrr_v, rows_v, sem_g, sem_a, sem_m):
    c = lax.axis_index("c")
    s = lax.axis_index("s")
    nch = jnp.where(c == 0, CH0, CH1)
    base = jnp.where(c == 0, s * CH0, NS * CH0 + s * CH1)
    hi = _hi16()

    def fill_body(i, carry):
        for f in range(D // LANES):
            rows_v[0][i, pl.ds(f * LANES, LANES)] = jnp.zeros((LANES,), jnp.float32)
        return carry
    lax.fori_loop(0, K2, fill_body, 0)

    for q in range(ST // K2):
        pltpu.async_copy(rows_v[0], acc_sh.at[pl.ds(s * ST + q * K2, K2)], sem_a[0])
    pltpu.async_copy(rows_v[0].at[pl.ds(0, ST % K2)],
                     acc_sh.at[pl.ds(s * ST + (ST // K2) * K2, ST % K2)], sem_a[0])
    for q in range(ST // K2):
        pltpu.make_async_copy(table.at[pl.ds(0, K2)], rows_v[0], sem_a[0]).wait()
    pltpu.make_async_copy(table.at[pl.ds(0, ST % K2)],
                          rows_v[0].at[pl.ds(0, ST % K2)], sem_a[0]).wait()

    plsc.subcore_barrier()

    def stage_issue(j, m):
        pltpu.async_copy(meta3.at[base + j], meta_v[m], sem_m[m])
        pltpu.async_copy(wrr.at[pl.ds((base + j) * K, K)], wrr_v[m], sem_m[m])

    def stage_wait(m):
        pltpu.make_async_copy(meta3.at[0], meta_v[m], sem_m[m]).wait()
        pltpu.make_async_copy(wrr.at[pl.ds(0, K)], wrr_v[m], sem_m[m]).wait()

    def deg_build(m, b):
        def deg_body(e, inner):
            gsp = wrr_v[m][e, pl.ds(LANES, LANES)]
            for f in range(D // LANES):
                cf = hi + float(8 * f)
                rows_v[b][K + e, pl.ds(f * LANES, LANES)] = jnp.where(
                    gsp == cf, 1.0, 0.0)
            return inner
        lax.fori_loop(0, K, deg_body, 0)

    def gather(m, b):
        pltpu.async_copy(table.at[meta_v[m].at[0, pl.ds(0, K)]],
                         rows_v[b].at[pl.ds(0, K)], sem_g[b])

    stage_issue(0, 0)
    stage_wait(0)
    deg_build(0, 0)
    gather(0, 0)
    stage_issue(1, 1)

    def quad_body(jj, carry):
        for u in range(4):
            j = jj * 4 + u
            b = u % 2
            nb = 1 - b
            m1 = (u + 1) % 4
            m2 = (u + 2) % 4

            @pl.when(j + 2 < nch)
            def _():
                stage_issue(j + 2, m2)

            @pl.when(j >= 1)
            def _():
                pltpu.make_async_copy(
                    table.at[pl.ds(0, K2)], rows_v[nb], sem_a[nb]).wait()

            @pl.when(j + 1 < nch)
            def _():
                stage_wait(m1)
                gather(m1, nb)

            pltpu.make_async_copy(
                table.at[pl.ds(0, K)], rows_v[b].at[pl.ds(0, K)], sem_g[b]).wait()

            def fused_body(e, inner):
                wsp = wrr_v[u][e, pl.ds(0, LANES)]
                for f in range(D // LANES):
                    sl = pl.ds(f * LANES, LANES)
                    rows_v[b][e, sl] = rows_v[b][e, sl] * wsp
                gsp = wrr_v[m1][e, pl.ds(LANES, LANES)]
                for f in range(D // LANES):
                    cf = hi + float(8 * f)
                    rows_v[nb][K + e, pl.ds(f * LANES, LANES)] = jnp.where(
                        gsp == cf, 1.0, 0.0)
                return inner
            lax.fori_loop(0, K, fused_body, 0)

            pltpu.async_copy(rows_v[b], acc_sh.at[meta_v[u].at[1]], sem_a[b],
                             add=True)
        return carry
    lax.fori_loop(0, nch // 4, quad_body, 0)

    pltpu.make_async_copy(table.at[pl.ds(0, K2)], rows_v[1], sem_a[1]).wait()

    plsc.subcore_barrier()

    NQ = ST // K2
    for q in range(NQ + 1):
        b = q % 2
        rows = K2 if q < NQ else ST % K2
        sl = pl.ds(s * ST + q * K2, rows)
        buf = rows_v[b] if rows == K2 else rows_v[b].at[pl.ds(0, rows)]
        if q >= 2:
            prev = K2 if q - 2 < NQ else ST % K2
            pbuf = rows_v[b] if prev == K2 else rows_v[b].at[pl.ds(0, prev)]
            pltpu.make_async_copy(table.at[pl.ds(0, prev)], pbuf, sem_a[b]).wait()
        pltpu.sync_copy(acc_sh.at[sl], buf)
        pltpu.async_copy(buf, accp.at[c, sl], sem_a[b])
    for q in [NQ - 1, NQ]:
        b = q % 2
        rows = K2 if q < NQ else ST % K2
        fbuf = rows_v[b] if rows == K2 else rows_v[b].at[pl.ds(0, rows)]
        pltpu.make_async_copy(table.at[pl.ds(0, rows)], fbuf, sem_a[b]).wait()


def kernel(x, edge_index, edge_type, num_nodes, edge_weight, W, W0):
    x = x.astype(jnp.float32)

    Wall = jnp.concatenate([W0[None], W], axis=0)
    BM = 1000
    table = pl.pallas_call(
        _mm_body,
        grid=(N // BM,),
        in_specs=[
            pl.BlockSpec((BM, D), lambda b: (b, 0)),
            pl.BlockSpec((R + 1, D, D), lambda b: (0, 0, 0)),
        ],
        out_specs=pl.BlockSpec((R + 1, BM, D), lambda b: (0, b, 0)),
        out_shape=jax.ShapeDtypeStruct((R + 1, N, D), jnp.float32),
    )(x, Wall).reshape((R + 1) * N, D)

    pad = EP - E
    src_p = jnp.pad(edge_index[0].astype(jnp.int32), (0, pad)).reshape(G, K)
    et_p = jnp.pad(edge_type.astype(jnp.int32), (0, pad)).reshape(G, K)
    dst2 = jnp.pad(edge_index[1].astype(jnp.int32), (0, pad),
                   constant_values=N).reshape(G, K)
    w2 = jnp.pad(edge_weight.astype(jnp.float32), (0, pad)).reshape(G, K)

    EB = 1024
    meta3, wrr3 = pl.pallas_call(
        _idx_body,
        grid=(G // EB,),
        in_specs=[
            pl.BlockSpec((EB, K), lambda i: (i, 0)),
            pl.BlockSpec((EB, K), lambda i: (i, 0)),
            pl.BlockSpec((EB, K), lambda i: (i, 0)),
            pl.BlockSpec((EB, K), lambda i: (i, 0)),
        ],
        out_specs=[
            pl.BlockSpec((EB, 2, K2), lambda i: (i, 0, 0)),
            pl.BlockSpec((EB, K, 2 * LANES), lambda i: (i, 0, 0)),
        ],
        out_shape=[
            jax.ShapeDtypeStruct((G, 2, K2), jnp.int32),
            jax.ShapeDtypeStruct((G, K, 2 * LANES), jnp.float32),
        ],
    )(src_p, et_p, dst2, w2)
    wrr = wrr3.reshape(EP, 2 * LANES)

    mesh = plsc.VectorSubcoreMesh(core_axis_name="c", subcore_axis_name="s",
                                  num_cores=NC, num_subcores=NS)
    accp = pl.kernel(
        _sc_body,
        out_type=jax.ShapeDtypeStruct((NC, NT, D), jnp.float32),
        mesh=mesh,
        scratch_types=[
            pltpu.VMEM_SHARED((NT, D), jnp.float32),
            [pltpu.VMEM((2, K2), jnp.int32)] * 4,
            [pltpu.VMEM((K, 2 * LANES), jnp.float32)] * 4,
            [pltpu.VMEM((K2, D), jnp.float32)] * 2,
            [pltpu.SemaphoreType.DMA] * 2,
            [pltpu.SemaphoreType.DMA] * 2,
            [pltpu.SemaphoreType.DMA] * 4,
        ],
    )(table, meta3, wrr)

    degx0 = accp[0, NPAD:NPAD + NPAD // 64].reshape(NPAD // 64, 64, 2)[
        :, :, 0:1].reshape(NPAD, 1)
    degx1 = accp[1, NPAD:NPAD + NPAD // 64].reshape(NPAD // 64, 64, 2)[
        :, :, 0:1].reshape(NPAD, 1)

    BF = 2000
    out = pl.pallas_call(
        _finish_body,
        grid=(N // BF,),
        in_specs=[
            pl.BlockSpec((BF, D), lambda b: (b, 0)),
            pl.BlockSpec((1, BF, D), lambda b: (0, b, 0)),
            pl.BlockSpec((1, BF, D), lambda b: (1, b, 0)),
            pl.BlockSpec((BF, 1), lambda b: (b, 0)),
            pl.BlockSpec((BF, 1), lambda b: (b, 0)),
        ],
        out_specs=pl.BlockSpec((BF, D), lambda b: (b, 0)),
        out_shape=jax.ShapeDtypeStruct((N, D), jnp.float32),
    )(table, accp, accp, degx0, degx1)
    return out


# speedup vs baseline: 1.2042x; 1.0124x over previous
import jax
import jax.numpy as jnp
from jax import lax
from jax.experimental import pallas as pl
from jax.experimental.pallas import tpu as pltpu
from jax.experimental.pallas import tpu_sc as plsc

N = 10000
E = 320000
D = 128
R = 8

NC, NS, LANES = 2, 16, 16
NW = NC * NS
K = 32
K2 = 2 * K
EPT = 10240
CH = EPT // K
CH0 = 520
CH1 = 2 * CH - CH0
EP = EPT * NW
G = EP // K
NPAD = 10240
DR = 256
NT = NPAD + DR
ST = NT // NS
BN = 1000
NB = N // BN


def _mm_body(x_ref, w_ref, o_ref):
    for r in range(R + 1):
        o_ref[r] = jnp.dot(x_ref[...], w_ref[r],
                           preferred_element_type=jnp.float32)


def _idx_body(src_ref, et_ref, dst_ref, w_ref, meta_ref, wrr_ref):
    dst = dst_ref[...]
    meta_ref[:, 0, 0:K] = (et_ref[...] + 1) * N + src_ref[...]
    meta_ref[:, 0, K:K2] = jnp.zeros(dst.shape, jnp.int32)
    meta_ref[:, 1, 0:K] = dst
    meta_ref[:, 1, K:K2] = NPAD + (dst >> 6)
    wrr_ref[:, :, 0:LANES] = jnp.broadcast_to(
        w_ref[...][:, :, None], (dst.shape[0], K, LANES))
    wrr_ref[:, :, LANES:] = jnp.broadcast_to(
        (dst & 63).astype(jnp.float32)[:, :, None], (dst.shape[0], K, LANES))


def _finish_body(root_ref, a0_ref, a1_ref, d0_ref, d1_ref, o_ref):
    deg = jnp.maximum(d0_ref[...] + d1_ref[...], 1.0)
    o_ref[...] = (root_ref[...] + a0_ref[0] + a1_ref[0]) / deg


def _hi16():
    return lax.shift_right_logical(
        lax.iota(jnp.int32, 16), jnp.full((16,), 1, jnp.int32)
    ).astype(jnp.float32)


def _sc_body(table, meta3, wrr, accp,
             acc_sh, meta_v, wrr_v, rows_v, sem_g, sem_a, sem_m):
    c = lax.axis_index("c")
    s = lax.axis_index("s")
    nch = jnp.where(c == 0, CH0, CH1)
    base = jnp.where(c == 0, s * CH0, NS * CH0 + s * CH1)
    hi = _hi16()

    def fill_body(i, carry):
        for f in range(D // LANES):
            rows_v[0][i, pl.ds(f * LANES, LANES)] = jnp.zeros((LANES,), jnp.float32)
        return carry
    lax.fori_loop(0, K2, fill_body, 0)

    for q in range(ST // K2):
        pltpu.async_copy(rows_v[0], acc_sh.at[pl.ds(s * ST + q * K2, K2)], sem_a[0])
    pltpu.async_copy(rows_v[0].at[pl.ds(0, ST % K2)],
                     acc_sh.at[pl.ds(s * ST + (ST // K2) * K2, ST % K2)], sem_a[0])
    for q in range(ST // K2):
        pltpu.make_async_copy(table.at[pl.ds(0, K2)], rows_v[0], sem_a[0]).wait()
    pltpu.make_async_copy(table.at[pl.ds(0, ST % K2)],
                          rows_v[0].at[pl.ds(0, ST % K2)], sem_a[0]).wait()

    plsc.subcore_barrier()

    def stage_issue(j, m):
        pltpu.async_copy(meta3.at[base + j], meta_v[m], sem_m[m])
        pltpu.async_copy(wrr.at[pl.ds((base + j) * K, K)], wrr_v[m], sem_m[m])

    def stage_wait(m):
        pltpu.make_async_copy(meta3.at[0], meta_v[m], sem_m[m]).wait()
        pltpu.make_async_copy(wrr.at[pl.ds(0, K)], wrr_v[m], sem_m[m]).wait()

    def deg_build(m, b):
        def deg_body(e, inner):
            gsp = wrr_v[m][e, pl.ds(LANES, LANES)]
            for f in range(D // LANES):
                cf = hi + float(8 * f)
                rows_v[b][K + e, pl.ds(f * LANES, LANES)] = jnp.where(
                    gsp == cf, 1.0, 0.0)
            return inner
        lax.fori_loop(0, K, deg_body, 0)

    def gather(m, b):
        pltpu.async_copy(table.at[meta_v[m].at[0, pl.ds(0, K)]],
                         rows_v[b].at[pl.ds(0, K)], sem_g[b])

    stage_issue(0, 0)
    stage_wait(0)
    deg_build(0, 0)
    gather(0, 0)
    stage_issue(1, 1)

    def quad_body(jj, carry):
        for u in range(4):
            j = jj * 4 + u
            b = u % 2
            nb = 1 - b
            m1 = (u + 1) % 4
            m2 = (u + 2) % 4

            @pl.when(j + 2 < nch)
            def _():
                stage_issue(j + 2, m2)

            @pl.when(j >= 1)
            def _():
                pltpu.make_async_copy(
                    table.at[pl.ds(0, K2)], rows_v[nb], sem_a[nb]).wait()

            @pl.when(j + 1 < nch)
            def _():
                stage_wait(m1)
                gather(m1, nb)

            pltpu.make_async_copy(
                table.at[pl.ds(0, K)], rows_v[b].at[pl.ds(0, K)], sem_g[b]).wait()

            def fused_body(e, inner):
                wsp = wrr_v[u][e, pl.ds(0, LANES)]
                for f in range(D // LANES):
                    sl = pl.ds(f * LANES, LANES)
                    rows_v[b][e, sl] = rows_v[b][e, sl] * wsp
                gsp = wrr_v[m1][e, pl.ds(LANES, LANES)]
                for f in range(D // LANES):
                    cf = hi + float(8 * f)
                    rows_v[nb][K + e, pl.ds(f * LANES, LANES)] = jnp.where(
                        gsp == cf, 1.0, 0.0)
                return inner
            lax.fori_loop(0, K, fused_body, 0)

            pltpu.async_copy(rows_v[b], acc_sh.at[meta_v[u].at[1]], sem_a[b],
                             add=True)
        return carry
    lax.fori_loop(0, nch // 4, quad_body, 0)

    pltpu.make_async_copy(table.at[pl.ds(0, K2)], rows_v[1], sem_a[1]).wait()

    plsc.subcore_barrier()

    NQ = ST // K2
    for q in range(NQ + 1):
        b = q % 2
        rows = K2 if q < NQ else ST % K2
        sl = pl.ds(s * ST + q * K2, rows)
        buf = rows_v[b] if rows == K2 else rows_v[b].at[pl.ds(0, rows)]
        if q >= 2:
            prev = K2 if q - 2 < NQ else ST % K2
            pbuf = rows_v[b] if prev == K2 else rows_v[b].at[pl.ds(0, prev)]
            pltpu.make_async_copy(table.at[pl.ds(0, prev)], pbuf, sem_a[b]).wait()
        pltpu.sync_copy(acc_sh.at[sl], buf)
        pltpu.async_copy(buf, accp.at[c, sl], sem_a[b])
    for q in [NQ - 1, NQ]:
        b = q % 2
        rows = K2 if q < NQ else ST % K2
        fbuf = rows_v[b] if rows == K2 else rows_v[b].at[pl.ds(0, rows)]
        pltpu.make_async_copy(table.at[pl.ds(0, rows)], fbuf, sem_a[b]).wait()


def kernel(x, edge_index, edge_type, num_nodes, edge_weight, W, W0):
    x = x.astype(jnp.float32)

    Wall = jnp.concatenate([W0[None], W], axis=0)
    BM = 1000
    table = pl.pallas_call(
        _mm_body,
        grid=(N // BM,),
        in_specs=[
            pl.BlockSpec((BM, D), lambda b: (b, 0)),
            pl.BlockSpec((R + 1, D, D), lambda b: (0, 0, 0)),
        ],
        out_specs=pl.BlockSpec((R + 1, BM, D), lambda b: (0, b, 0)),
        out_shape=jax.ShapeDtypeStruct((R + 1, N, D), jnp.float32),
    )(x, Wall).reshape((R + 1) * N, D)

    pad = EP - E
    src_p = jnp.pad(edge_index[0].astype(jnp.int32), (0, pad)).reshape(G, K)
    et_p = jnp.pad(edge_type.astype(jnp.int32), (0, pad)).reshape(G, K)
    dst2 = jnp.pad(edge_index[1].astype(jnp.int32), (0, pad),
                   constant_values=N).reshape(G, K)
    w2 = jnp.pad(edge_weight.astype(jnp.float32), (0, pad)).reshape(G, K)

    EB = 1024
    meta3, wrr3 = pl.pallas_call(
        _idx_body,
        grid=(G // EB,),
        in_specs=[
            pl.BlockSpec((EB, K), lambda i: (i, 0)),
            pl.BlockSpec((EB, K), lambda i: (i, 0)),
            pl.BlockSpec((EB, K), lambda i: (i, 0)),
            pl.BlockSpec((EB, K), lambda i: (i, 0)),
        ],
        out_specs=[
            pl.BlockSpec((EB, 2, K2), lambda i: (i, 0, 0)),
            pl.BlockSpec((EB, K, 2 * LANES), lambda i: (i, 0, 0)),
        ],
        out_shape=[
            jax.ShapeDtypeStruct((G, 2, K2), jnp.int32),
            jax.ShapeDtypeStruct((G, K, 2 * LANES), jnp.float32),
        ],
    )(src_p, et_p, dst2, w2)
    wrr = wrr3.reshape(EP, 2 * LANES)

    mesh = plsc.VectorSubcoreMesh(core_axis_name="c", subcore_axis_name="s",
                                  num_cores=NC, num_subcores=NS)
    accp = pl.kernel(
        _sc_body,
        out_type=jax.ShapeDtypeStruct((NC, NT, D), jnp.float32),
        mesh=mesh,
        scratch_types=[
            pltpu.VMEM_SHARED((NT, D), jnp.float32),
            [pltpu.VMEM((2, K2), jnp.int32)] * 4,
            [pltpu.VMEM((K, 2 * LANES), jnp.float32)] * 4,
            [pltpu.VMEM((K2, D), jnp.float32)] * 2,
            [pltpu.SemaphoreType.DMA] * 2,
            [pltpu.SemaphoreType.DMA] * 2,
            [pltpu.SemaphoreType.DMA] * 4,
        ],
    )(table, meta3, wrr)

    degx0 = accp[0, NPAD:NPAD + NPAD // 64].reshape(NPAD // 64, 64, 2)[
        :, :, 0:1].reshape(NPAD, 1)
    degx1 = accp[1, NPAD:NPAD + NPAD // 64].reshape(NPAD // 64, 64, 2)[
        :, :, 0:1].reshape(NPAD, 1)

    BF = 2000
    out = pl.pallas_call(
        _finish_body,
        grid=(N // BF,),
        in_specs=[
            pl.BlockSpec((BF, D), lambda b: (b, 0)),
            pl.BlockSpec((1, BF, D), lambda b: (0, b, 0)),
            pl.BlockSpec((1, BF, D), lambda b: (1, b, 0)),
            pl.BlockSpec((BF, 1), lambda b: (b, 0)),
            pl.BlockSpec((BF, 1), lambda b: (b, 0)),
        ],
        out_specs=pl.BlockSpec((BF, D), lambda b: (b, 0)),
        out_shape=jax.ShapeDtypeStruct((N, D), jnp.float32),
    )(table, accp, accp, degx0, degx1)
    return out
